# Initial kernel scaffold; baseline (speedup 1.0000x reference)
import jax
import jax.numpy as jnp
from jax.experimental import pallas as pl


def kernel(x, adj, row_ptr, W1, b1, W2, b2, beta):
    raise NotImplementedError("write your pallas kernel here")


# speedup vs baseline: 3.9266x; 3.9266x over previous
import functools

import jax
import jax.numpy as jnp
from jax import lax
from jax.experimental import pallas as pl
from jax.experimental.pallas import tpu as pltpu
from jax.experimental.pallas import tpu_sc as plsc

NC = 2
NS = 16


def _dense_in_body(x_ref, w1_ref, b1_ref, h_ref, hn_ref):
    h = lax.dot_general(x_ref[...], w1_ref[...], (((1,), (1,)), ((), ())),
                        preferred_element_type=jnp.float32,
                        precision=lax.Precision.HIGHEST)
    h = jnp.maximum(h + b1_ref[...], 0.0)
    nrm = jnp.sqrt(jnp.sum(h * h, axis=1, keepdims=True))
    h_ref[...] = h
    hn_ref[...] = h / jnp.maximum(nrm, 1e-12)


def _dsum_body(a_ref, o_ref):
    o_ref[...] = jnp.sum(a_ref[...], axis=0, keepdims=True)


def _combine_body(accf_ref, accd_ref, h_ref, hn_ref):
    a = accf_ref[0] + accf_ref[1]
    den = accd_ref[...]
    h = jnp.maximum(a / (den + 1e-16), 0.0)
    nrm = jnp.sqrt(jnp.sum(h * h, axis=1, keepdims=True))
    h_ref[...] = h
    hn_ref[...] = h / jnp.maximum(nrm, 1e-12)


def _dense_out_body(h_ref, w2_ref, b2_ref, o_ref):
    o = lax.dot_general(h_ref[...], w2_ref[...], (((1,), (1,)), ((), ())),
                        preferred_element_type=jnp.float32,
                        precision=lax.Precision.HIGHEST)
    o_ref[...] = o + b2_ref[...]


def _make_sc_layer(N, D, E):
    NW = NC * NS
    assert E % NW == 0
    EPW = E // NW
    B = 16
    assert EPW % B == 0
    NBLK = EPW // B
    ZR = 128
    NPAD = ((N + NS * ZR - 1) // (NS * ZR)) * (NS * ZR)
    RPT = NPAD // NS
    NZ = RPT // ZR
    ND = NPAD // 16
    DPT = ND // NS
    assert DPT % 8 == 0
    KD = D // 16
    NG = B // 16

    mesh = plsc.VectorSubcoreMesh(core_axis_name="c", subcore_axis_name="s",
                                  num_cores=NC, num_subcores=NS)

    @functools.partial(
        pl.kernel,
        out_type=[pltpu.HBM((NC, NPAD, D), jnp.float32),
                  pltpu.HBM((NC, NS, NPAD), jnp.float32)],
        mesh=mesh,
        scratch_types=[
            pltpu.VMEM((B,), jnp.int32),
            pltpu.VMEM((B,), jnp.int32),
            pltpu.VMEM((B,), jnp.int32),
            pltpu.VMEM((B, D), jnp.float32),
            pltpu.VMEM((B, D), jnp.float32),
            pltpu.VMEM((B, D), jnp.float32),
            pltpu.VMEM((B, D), jnp.float32),
            pltpu.VMEM((NPAD,), jnp.float32),
            pltpu.VMEM((32,), jnp.float32),
            pltpu.VMEM((ZR, 128), jnp.float32),
            pltpu.VMEM_SHARED((NPAD, D), jnp.float32),
            pltpu.SemaphoreType.DMA,
            pltpu.SemaphoreType.DMA,
            pltpu.SemaphoreType.DMA,
        ],
    )
    def sc_layer(hn_hbm, h_hbm, rows_hbm, cols_hbm, rowsd_hbm,
                 cst_hbm, accf_hbm, accd_hbm,
                 ridx, cidx, didx, rbuf, cbuf, hbuf, ybuf, dacc, cst,
                 zbuf, accf, sem1, sem2, sem3):
        cid = lax.axis_index("c")
        sid = lax.axis_index("s")
        wid = sid * NC + cid

        pltpu.sync_copy(cst_hbm, cst)

        zvec = jnp.zeros((16,), jnp.float32)

        @pl.loop(0, ZR)
        def _(r):
            for k in range(8):
                zbuf[r, pl.ds(k * 16, 16)] = zvec

        @pl.loop(0, NPAD // 16)
        def _(r):
            dacc[pl.ds(r * 16, 16)] = zvec

        base_row = sid * RPT
        for q in range(NZ):
            pltpu.sync_copy(zbuf, accf.at[pl.ds(base_row + q * ZR, ZR)])
        plsc.subcore_barrier()

        beta_v = cst[pl.ds(0, 16)]
        babs_v = cst[pl.ds(16, 16)]
        io = lax.iota(jnp.int32, 16)
        iof = io.astype(jnp.float32)

        ebase = wid * EPW

        @pl.loop(0, NBLK)
        def _(b):
            base = ebase + b * B
            pltpu.sync_copy(rows_hbm.at[pl.ds(base, B)], ridx)
            pltpu.sync_copy(cols_hbm.at[pl.ds(base, B)], cidx)
            pltpu.sync_copy(rowsd_hbm.at[pl.ds(base, B)], didx)
            cp1 = pltpu.async_copy(hn_hbm.at[ridx], rbuf, sem1)
            cp2 = pltpu.async_copy(hn_hbm.at[cidx], cbuf, sem2)
            cp3 = pltpu.async_copy(h_hbm.at[cidx], hbuf, sem3)
            cp1.wait()
            cp2.wait()
            cp3.wait()

            @pl.loop(0, NG)
            def _(j):
                cosv = jnp.zeros((16,), jnp.float32)
                for r in range(16):
                    e = j * 16 + r
                    s = rbuf[e, pl.ds(0, 16)] * cbuf[e, pl.ds(0, 16)]
                    for k in range(1, KD):
                        s = s + rbuf[e, pl.ds(k * 16, 16)] * cbuf[e, pl.ds(k * 16, 16)]
                    c = s[0]
                    for i in range(1, 16):
                        c = c + s[i]
                    cosv = jnp.where(io == r, c, cosv)
                w16 = jnp.exp(beta_v * cosv - babs_v)
                offf = (ridx[pl.ds(j * 16, 16)] & 15).astype(jnp.float32)
                didx16 = didx[pl.ds(j * 16, 16)]
                for r in range(16):
                    e = j * 16 + r
                    wv = jnp.full((16,), w16[r], jnp.float32)
                    for k in range(KD):
                        ybuf[e, pl.ds(k * 16, 16)] = wv * hbuf[e, pl.ds(k * 16, 16)]
                    off_r = jnp.full((16,), offf[r], jnp.float32)
                    dbase = didx16[r] * 16
                    dacc[pl.ds(dbase, 16)] = (dacc[pl.ds(dbase, 16)]
                                              + jnp.where(iof == off_r, wv, zvec))

            pltpu.sync_copy(ybuf, accf.at[ridx], add=True)

        plsc.subcore_barrier()
        for q in range(NZ):
            pltpu.sync_copy(accf.at[pl.ds(base_row + q * ZR, ZR)], zbuf)
            pltpu.sync_copy(zbuf, accf_hbm.at[cid, pl.ds(base_row + q * ZR, ZR)])
        pltpu.sync_copy(dacc, accd_hbm.at[cid, sid])

    return sc_layer


def kernel(x, adj, row_ptr, W1, b1, W2, b2, beta):
    N, D = x.shape
    E = adj.shape[1]
    RB = 2000
    assert N % RB == 0

    cols = adj[0]
    rows = adj[1]
    rowsd = jax.lax.shift_right_logical(rows, 4)
    consts = jnp.concatenate([jnp.broadcast_to(beta, (16,)),
                              jnp.broadcast_to(jnp.abs(beta), (16,))]
                             ).astype(jnp.float32)

    dense_in = pl.pallas_call(
        _dense_in_body,
        grid=(N // RB,),
        in_specs=[pl.BlockSpec((RB, D), lambda i: (i, 0)),
                  pl.BlockSpec((D, D), lambda i: (0, 0)),
                  pl.BlockSpec((1, D), lambda i: (0, 0))],
        out_specs=[pl.BlockSpec((RB, D), lambda i: (i, 0)),
                   pl.BlockSpec((RB, D), lambda i: (i, 0))],
        out_shape=[jax.ShapeDtypeStruct((N, D), jnp.float32),
                   jax.ShapeDtypeStruct((N, D), jnp.float32)],
    )

    combine = pl.pallas_call(
        _combine_body,
        grid=(N // RB,),
        in_specs=[pl.BlockSpec((NC, RB, D), lambda i: (0, i, 0)),
                  pl.BlockSpec((RB, 1), lambda i: (i, 0))],
        out_specs=[pl.BlockSpec((RB, D), lambda i: (i, 0)),
                   pl.BlockSpec((RB, D), lambda i: (i, 0))],
        out_shape=[jax.ShapeDtypeStruct((N, D), jnp.float32),
                   jax.ShapeDtypeStruct((N, D), jnp.float32)],
    )

    dense_out = pl.pallas_call(
        _dense_out_body,
        grid=(N // RB,),
        in_specs=[pl.BlockSpec((RB, D), lambda i: (i, 0)),
                  pl.BlockSpec((D, D), lambda i: (0, 0)),
                  pl.BlockSpec((1, D), lambda i: (0, 0))],
        out_specs=pl.BlockSpec((RB, D), lambda i: (i, 0)),
        out_shape=jax.ShapeDtypeStruct((N, D), jnp.float32),
    )

    sc_layer = _make_sc_layer(N, D, E)
    NPAD = 2048 * ((N + 2047) // 2048)
    dsum = pl.pallas_call(
        _dsum_body,
        grid=(1,),
        in_specs=[pl.BlockSpec((NC * NS, NPAD), lambda i: (0, 0))],
        out_specs=pl.BlockSpec((1, NPAD), lambda i: (0, 0)),
        out_shape=jax.ShapeDtypeStruct((1, NPAD), jnp.float32),
    )

    h, hn = dense_in(x, W1, b1.reshape(1, D))
    for _ in range(4):
        accf, accd = sc_layer(hn, h, rows, cols, rowsd, consts)
        den = dsum(accd.reshape(NC * NS, NPAD)).reshape(NPAD, 1)[:N]
        h, hn = combine(accf, den)
    return dense_out(h, W2, b2.reshape(1, D))


# speedup vs baseline: 8.6318x; 2.1983x over previous
import functools

import jax
import jax.numpy as jnp
from jax import lax
from jax.experimental import pallas as pl
from jax.experimental.pallas import tpu as pltpu
from jax.experimental.pallas import tpu_sc as plsc

NC = 2
NS = 16


def _dense_in_body(x_ref, w1_ref, b1_ref, h_ref, hn_ref):
    h = lax.dot_general(x_ref[...], w1_ref[...], (((1,), (1,)), ((), ())),
                        preferred_element_type=jnp.float32,
                        precision=lax.Precision.HIGHEST)
    h = jnp.maximum(h + b1_ref[...], 0.0)
    nrm = jnp.sqrt(jnp.sum(h * h, axis=1, keepdims=True))
    h_ref[...] = h
    hn_ref[...] = h / jnp.maximum(nrm, 1e-12)


def _dsum_body(a_ref, o_ref):
    o_ref[...] = jnp.sum(a_ref[...], axis=0, keepdims=True)


def _combine_body(accf_ref, accd_ref, h_ref, hn_ref):
    a = accf_ref[0] + accf_ref[1]
    den = accd_ref[...]
    h = jnp.maximum(a / (den + 1e-16), 0.0)
    nrm = jnp.sqrt(jnp.sum(h * h, axis=1, keepdims=True))
    h_ref[...] = h
    hn_ref[...] = h / jnp.maximum(nrm, 1e-12)


def _dense_out_body(h_ref, w2_ref, b2_ref, o_ref):
    o = lax.dot_general(h_ref[...], w2_ref[...], (((1,), (1,)), ((), ())),
                        preferred_element_type=jnp.float32,
                        precision=lax.Precision.HIGHEST)
    o_ref[...] = o + b2_ref[...]


def _make_sc_layer(N, D, E):
    NW = NC * NS
    B = 64
    assert E % B == 0
    NBT = E // B
    FULLN = (NBT + NW - 1) // NW
    LASTN = NBT - (NW - 1) * FULLN
    assert 0 < LASTN <= FULLN
    ZR = 64
    NPAD = ((N + NS * ZR - 1) // (NS * ZR)) * (NS * ZR)
    RPT = NPAD // NS
    NZ = RPT // ZR
    ND = NPAD // 16
    DPT = ND // NS
    assert DPT % 8 == 0
    KD = D // 16
    NG = B // 16

    mesh = plsc.VectorSubcoreMesh(core_axis_name="c", subcore_axis_name="s",
                                  num_cores=NC, num_subcores=NS)

    @functools.partial(
        pl.kernel,
        out_type=[pltpu.HBM((NC, NPAD, D), jnp.float32),
                  pltpu.HBM((NC, NS, NPAD), jnp.float32)],
        mesh=mesh,
        scratch_types=[
            pltpu.VMEM((B,), jnp.int32),
            [pltpu.VMEM((16,), jnp.int32) for _ in range(B // 16)],
            pltpu.VMEM((B,), jnp.int32),
            pltpu.VMEM((B,), jnp.int32),
            pltpu.VMEM((B, D), jnp.float32),
            pltpu.VMEM((B, D), jnp.float32),
            pltpu.VMEM((B, D), jnp.float32),
            pltpu.VMEM((NPAD,), jnp.float32),
            pltpu.VMEM((32,), jnp.float32),
            pltpu.VMEM((ZR, 128), jnp.float32),
            pltpu.VMEM_SHARED((NPAD, D), jnp.float32),
            pltpu.SemaphoreType.DMA,
            pltpu.SemaphoreType.DMA,
            pltpu.SemaphoreType.DMA,
            pltpu.SemaphoreType.DMA,
        ],
    )
    def sc_layer(hn_hbm, h_hbm, rows_hbm, cols_hbm, rowsd_hbm,
                 cst_hbm, accf_hbm, accd_hbm,
                 ridx, ridxg, cidx, didx, rbuf, cbuf, hbuf, dacc, cst,
                 zbuf, accf, semi, sem1, sem2, sem3):
        cid = lax.axis_index("c")
        sid = lax.axis_index("s")
        wid = sid * NC + cid

        pltpu.sync_copy(cst_hbm, cst)

        zvec = jnp.zeros((16,), jnp.float32)

        @pl.loop(0, ZR)
        def _(r):
            for k in range(8):
                zbuf[r, pl.ds(k * 16, 16)] = zvec

        @pl.loop(0, NPAD // 16)
        def _(r):
            dacc[pl.ds(r * 16, 16)] = zvec

        base_row = sid * RPT
        for q in range(NZ):
            pltpu.sync_copy(zbuf, accf.at[pl.ds(base_row + q * ZR, ZR)])
        plsc.subcore_barrier()

        beta_v = cst[pl.ds(0, 16)]
        babs_v = cst[pl.ds(16, 16)]
        io = lax.iota(jnp.int32, 16)
        iof = io.astype(jnp.float32)

        ebase = wid * FULLN * B
        nblk = jnp.where(wid == NW - 1, LASTN, FULLN)

        @pl.loop(0, nblk)
        def _(b):
            base = ebase + b * B
            cps = [pltpu.async_copy(rows_hbm.at[pl.ds(base, B)], ridx, semi),
                   pltpu.async_copy(cols_hbm.at[pl.ds(base, B)], cidx, semi),
                   pltpu.async_copy(rowsd_hbm.at[pl.ds(base, B)], didx, semi)]
            for g in range(B // 16):
                cps.append(pltpu.async_copy(
                    rows_hbm.at[pl.ds(base + g * 16, 16)], ridxg[g], semi))
            for cp in cps:
                cp.wait()
            cp1 = pltpu.async_copy(hn_hbm.at[ridx], rbuf, sem1)
            cp2 = pltpu.async_copy(hn_hbm.at[cidx], cbuf, sem2)
            cp3 = pltpu.async_copy(h_hbm.at[cidx], hbuf, sem3)
            cp1.wait()
            cp2.wait()
            cp3.wait()

            @pl.loop(0, NG)
            def _(j):
                cosv = jnp.zeros((16,), jnp.float32)
                for r in range(16):
                    e = j * 16 + r
                    s = rbuf[e, pl.ds(0, 16)] * cbuf[e, pl.ds(0, 16)]
                    for k in range(1, KD):
                        s = s + rbuf[e, pl.ds(k * 16, 16)] * cbuf[e, pl.ds(k * 16, 16)]
                    c = s[0]
                    for i in range(1, 16):
                        c = c + s[i]
                    cosv = jnp.where(io == r, c, cosv)
                w16 = jnp.exp(beta_v * cosv - babs_v)
                offf = (ridx[pl.ds(j * 16, 16)] & 15).astype(jnp.float32)
                didx16 = didx[pl.ds(j * 16, 16)]
                for r in range(16):
                    e = j * 16 + r
                    wv = jnp.full((16,), w16[r], jnp.float32)
                    for k in range(KD):
                        hbuf[e, pl.ds(k * 16, 16)] = wv * hbuf[e, pl.ds(k * 16, 16)]
                    off_r = jnp.full((16,), offf[r], jnp.float32)
                    dbase = didx16[r] * 16
                    dacc[pl.ds(dbase, 16)] = (dacc[pl.ds(dbase, 16)]
                                              + jnp.where(iof == off_r, wv, zvec))

            for g in range(B // 16):
                pltpu.sync_copy(hbuf.at[pl.ds(g * 16, 16)],
                                accf.at[ridxg[g]], add=True)

        plsc.subcore_barrier()
        for q in range(NZ):
            pltpu.sync_copy(accf.at[pl.ds(base_row + q * ZR, ZR)], zbuf)
            pltpu.sync_copy(zbuf, accf_hbm.at[cid, pl.ds(base_row + q * ZR, ZR)])
        pltpu.sync_copy(dacc, accd_hbm.at[cid, sid])

    return sc_layer


def kernel(x, adj, row_ptr, W1, b1, W2, b2, beta):
    N, D = x.shape
    E = adj.shape[1]
    RB = 2000
    assert N % RB == 0

    cols = adj[0]
    rows = adj[1]
    rowsd = jax.lax.shift_right_logical(rows, 4)
    consts = jnp.concatenate([jnp.broadcast_to(beta, (16,)),
                              jnp.broadcast_to(jnp.abs(beta), (16,))]
                             ).astype(jnp.float32)

    dense_in = pl.pallas_call(
        _dense_in_body,
        grid=(N // RB,),
        in_specs=[pl.BlockSpec((RB, D), lambda i: (i, 0)),
                  pl.BlockSpec((D, D), lambda i: (0, 0)),
                  pl.BlockSpec((1, D), lambda i: (0, 0))],
        out_specs=[pl.BlockSpec((RB, D), lambda i: (i, 0)),
                   pl.BlockSpec((RB, D), lambda i: (i, 0))],
        out_shape=[jax.ShapeDtypeStruct((N, D), jnp.float32),
                   jax.ShapeDtypeStruct((N, D), jnp.float32)],
    )

    combine = pl.pallas_call(
        _combine_body,
        grid=(N // RB,),
        in_specs=[pl.BlockSpec((NC, RB, D), lambda i: (0, i, 0)),
                  pl.BlockSpec((RB, 1), lambda i: (i, 0))],
        out_specs=[pl.BlockSpec((RB, D), lambda i: (i, 0)),
                   pl.BlockSpec((RB, D), lambda i: (i, 0))],
        out_shape=[jax.ShapeDtypeStruct((N, D), jnp.float32),
                   jax.ShapeDtypeStruct((N, D), jnp.float32)],
    )

    dense_out = pl.pallas_call(
        _dense_out_body,
        grid=(N // RB,),
        in_specs=[pl.BlockSpec((RB, D), lambda i: (i, 0)),
                  pl.BlockSpec((D, D), lambda i: (0, 0)),
                  pl.BlockSpec((1, D), lambda i: (0, 0))],
        out_specs=pl.BlockSpec((RB, D), lambda i: (i, 0)),
        out_shape=jax.ShapeDtypeStruct((N, D), jnp.float32),
    )

    sc_layer = _make_sc_layer(N, D, E)
    NPAD = 2048 * ((N + 2047) // 2048)
    dsum = pl.pallas_call(
        _dsum_body,
        grid=(1,),
        in_specs=[pl.BlockSpec((NC * NS, NPAD), lambda i: (0, 0))],
        out_specs=pl.BlockSpec((1, NPAD), lambda i: (0, 0)),
        out_shape=jax.ShapeDtypeStruct((1, NPAD), jnp.float32),
    )

    h, hn = dense_in(x, W1, b1.reshape(1, D))
    for _ in range(4):
        accf, accd = sc_layer(hn, h, rows, cols, rowsd, consts)
        den = dsum(accd.reshape(NC * NS, NPAD)).reshape(NPAD, 1)[:N]
        h, hn = combine(accf, den)
    return dense_out(h, W2, b2.reshape(1, D))


# speedup vs baseline: 9.6859x; 1.1221x over previous
import functools

import jax
import jax.numpy as jnp
from jax import lax
from jax.experimental import pallas as pl
from jax.experimental.pallas import tpu as pltpu
from jax.experimental.pallas import tpu_sc as plsc

NC = 2
NS = 16


def _dense_in_body(x_ref, w1_ref, b1_ref, h_ref, hn_ref):
    h = lax.dot_general(x_ref[...], w1_ref[...], (((1,), (1,)), ((), ())),
                        preferred_element_type=jnp.float32,
                        precision=lax.Precision.HIGHEST)
    h = jnp.maximum(h + b1_ref[...], 0.0)
    nrm = jnp.sqrt(jnp.sum(h * h, axis=1, keepdims=True))
    h_ref[...] = h
    hn_ref[...] = h / jnp.maximum(nrm, 1e-12)


def _dsum_body(a_ref, o_ref):
    o_ref[...] = jnp.sum(a_ref[...], axis=0, keepdims=True)


def _combine_body(accf_ref, accd_ref, h_ref, hn_ref):
    a = accf_ref[0] + accf_ref[1]
    den = accd_ref[...]
    h = jnp.maximum(a / (den + 1e-16), 0.0)
    nrm = jnp.sqrt(jnp.sum(h * h, axis=1, keepdims=True))
    h_ref[...] = h
    hn_ref[...] = h / jnp.maximum(nrm, 1e-12)


def _dense_out_body(h_ref, w2_ref, b2_ref, o_ref):
    o = lax.dot_general(h_ref[...], w2_ref[...], (((1,), (1,)), ((), ())),
                        preferred_element_type=jnp.float32,
                        precision=lax.Precision.HIGHEST)
    o_ref[...] = o + b2_ref[...]


def _make_sc_layer(N, D, E):
    NW = NC * NS
    B = 64
    assert E % B == 0
    NBT = E // B
    FULLN = (NBT + NW - 1) // NW
    LASTN = NBT - (NW - 1) * FULLN
    assert 0 < LASTN <= FULLN
    ZR = 64
    NPAD = ((N + NS * ZR - 1) // (NS * ZR)) * (NS * ZR)
    RPT = NPAD // NS
    NZ = RPT // ZR
    ND = NPAD // 16
    DPT = ND // NS
    assert DPT % 8 == 0
    KD = D // 16
    NG = B // 16

    mesh = plsc.VectorSubcoreMesh(core_axis_name="c", subcore_axis_name="s",
                                  num_cores=NC, num_subcores=NS)

    @functools.partial(
        pl.kernel,
        out_type=[pltpu.HBM((NC, NPAD, D), jnp.float32),
                  pltpu.HBM((NC, NS, NPAD), jnp.float32)],
        mesh=mesh,
        scratch_types=[
            pltpu.VMEM((B,), jnp.int32),
            [pltpu.VMEM((16,), jnp.int32) for _ in range(B // 16)],
            pltpu.VMEM((B,), jnp.int32),
            pltpu.VMEM((B,), jnp.int32),
            pltpu.VMEM((B, D), jnp.float32),
            pltpu.VMEM((B, D), jnp.float32),
            pltpu.VMEM((B, D), jnp.float32),
            pltpu.VMEM((NPAD,), jnp.float32),
            pltpu.VMEM((32,), jnp.float32),
            pltpu.VMEM((ZR, 128), jnp.float32),
            pltpu.VMEM_SHARED((NPAD, D), jnp.float32),
            pltpu.SemaphoreType.DMA,
            pltpu.SemaphoreType.DMA,
            pltpu.SemaphoreType.DMA,
            pltpu.SemaphoreType.DMA,
            pltpu.SemaphoreType.DMA,
        ],
    )
    def sc_layer(hn_hbm, h_hbm, rows_hbm, cols_hbm, rowsd_hbm,
                 cst_hbm, accf_hbm, accd_hbm,
                 ridx, ridxg, cidx, didx, rbuf, cbuf, hbuf, dacc, cst,
                 zbuf, accf, semi, sem1, sem2, sem3, sems):
        cid = lax.axis_index("c")
        sid = lax.axis_index("s")
        wid = sid * NC + cid

        pltpu.sync_copy(cst_hbm, cst)

        zvec = jnp.zeros((16,), jnp.float32)

        @pl.loop(0, ZR)
        def _(r):
            for k in range(8):
                zbuf[r, pl.ds(k * 16, 16)] = zvec

        @pl.loop(0, NPAD // 16)
        def _(r):
            dacc[pl.ds(r * 16, 16)] = zvec

        base_row = sid * RPT
        for q in range(NZ):
            pltpu.sync_copy(zbuf, accf.at[pl.ds(base_row + q * ZR, ZR)])
        plsc.subcore_barrier()

        beta_v = cst[pl.ds(0, 16)]
        babs_v = cst[pl.ds(16, 16)]
        io = lax.iota(jnp.int32, 16)
        iof = io.astype(jnp.float32)

        ebase = wid * FULLN * B
        nblk = jnp.where(wid == NW - 1, LASTN, FULLN)

        @pl.loop(0, nblk)
        def _(b):
            base = ebase + b * B
            cps = [pltpu.async_copy(rows_hbm.at[pl.ds(base, B)], ridx, semi),
                   pltpu.async_copy(cols_hbm.at[pl.ds(base, B)], cidx, semi),
                   pltpu.async_copy(rowsd_hbm.at[pl.ds(base, B)], didx, semi)]
            for g in range(B // 16):
                cps.append(pltpu.async_copy(
                    rows_hbm.at[pl.ds(base + g * 16, 16)], ridxg[g], semi))
            for cp in cps:
                cp.wait()
            cp1 = pltpu.async_copy(hn_hbm.at[ridx], rbuf, sem1)
            cp2 = pltpu.async_copy(hn_hbm.at[cidx], cbuf, sem2)

            @pl.when(b > 0)
            def _():
                for g in range(B // 16):
                    pltpu.make_async_copy(
                        hbuf.at[pl.ds(g * 16, 16)], accf.at[ridxg[g]],
                        sems).wait()

            cp3 = pltpu.async_copy(h_hbm.at[cidx], hbuf, sem3)
            cp1.wait()
            cp2.wait()
            cp3.wait()

            @pl.loop(0, NG)
            def _(j):
                cosv = jnp.zeros((16,), jnp.float32)
                for r in range(16):
                    e = j * 16 + r
                    s = rbuf[e, pl.ds(0, 16)] * cbuf[e, pl.ds(0, 16)]
                    for k in range(1, KD):
                        s = s + rbuf[e, pl.ds(k * 16, 16)] * cbuf[e, pl.ds(k * 16, 16)]
                    c = s[0]
                    for i in range(1, 16):
                        c = c + s[i]
                    cosv = jnp.where(io == r, c, cosv)
                w16 = jnp.exp(beta_v * cosv - babs_v)
                offf = (ridx[pl.ds(j * 16, 16)] & 15).astype(jnp.float32)
                didx16 = didx[pl.ds(j * 16, 16)]
                for r in range(16):
                    e = j * 16 + r
                    wv = jnp.full((16,), w16[r], jnp.float32)
                    for k in range(KD):
                        hbuf[e, pl.ds(k * 16, 16)] = wv * hbuf[e, pl.ds(k * 16, 16)]
                    off_r = jnp.full((16,), offf[r], jnp.float32)
                    dbase = didx16[r] * 16
                    dacc[pl.ds(dbase, 16)] = (dacc[pl.ds(dbase, 16)]
                                              + jnp.where(iof == off_r, wv, zvec))

            for g in range(B // 16):
                pltpu.async_copy(hbuf.at[pl.ds(g * 16, 16)],
                                 accf.at[ridxg[g]], sems, add=True)

        for g in range(B // 16):
            pltpu.make_async_copy(
                hbuf.at[pl.ds(g * 16, 16)], accf.at[ridxg[g]], sems).wait()
        plsc.subcore_barrier()
        for q in range(NZ):
            pltpu.sync_copy(accf.at[pl.ds(base_row + q * ZR, ZR)], zbuf)
            pltpu.sync_copy(zbuf, accf_hbm.at[cid, pl.ds(base_row + q * ZR, ZR)])
        pltpu.sync_copy(dacc, accd_hbm.at[cid, sid])

    return sc_layer


def kernel(x, adj, row_ptr, W1, b1, W2, b2, beta):
    N, D = x.shape
    E = adj.shape[1]
    RB = 2000
    assert N % RB == 0

    cols = adj[0]
    rows = adj[1]
    rowsd = jax.lax.shift_right_logical(rows, 4)
    consts = jnp.concatenate([jnp.broadcast_to(beta, (16,)),
                              jnp.broadcast_to(jnp.abs(beta), (16,))]
                             ).astype(jnp.float32)

    dense_in = pl.pallas_call(
        _dense_in_body,
        grid=(N // RB,),
        in_specs=[pl.BlockSpec((RB, D), lambda i: (i, 0)),
                  pl.BlockSpec((D, D), lambda i: (0, 0)),
                  pl.BlockSpec((1, D), lambda i: (0, 0))],
        out_specs=[pl.BlockSpec((RB, D), lambda i: (i, 0)),
                   pl.BlockSpec((RB, D), lambda i: (i, 0))],
        out_shape=[jax.ShapeDtypeStruct((N, D), jnp.float32),
                   jax.ShapeDtypeStruct((N, D), jnp.float32)],
    )

    combine = pl.pallas_call(
        _combine_body,
        grid=(N // RB,),
        in_specs=[pl.BlockSpec((NC, RB, D), lambda i: (0, i, 0)),
                  pl.BlockSpec((RB, 1), lambda i: (i, 0))],
        out_specs=[pl.BlockSpec((RB, D), lambda i: (i, 0)),
                   pl.BlockSpec((RB, D), lambda i: (i, 0))],
        out_shape=[jax.ShapeDtypeStruct((N, D), jnp.float32),
                   jax.ShapeDtypeStruct((N, D), jnp.float32)],
    )

    dense_out = pl.pallas_call(
        _dense_out_body,
        grid=(N // RB,),
        in_specs=[pl.BlockSpec((RB, D), lambda i: (i, 0)),
                  pl.BlockSpec((D, D), lambda i: (0, 0)),
                  pl.BlockSpec((1, D), lambda i: (0, 0))],
        out_specs=pl.BlockSpec((RB, D), lambda i: (i, 0)),
        out_shape=jax.ShapeDtypeStruct((N, D), jnp.float32),
    )

    sc_layer = _make_sc_layer(N, D, E)
    NPAD = 2048 * ((N + 2047) // 2048)
    dsum = pl.pallas_call(
        _dsum_body,
        grid=(1,),
        in_specs=[pl.BlockSpec((NC * NS, NPAD), lambda i: (0, 0))],
        out_specs=pl.BlockSpec((1, NPAD), lambda i: (0, 0)),
        out_shape=jax.ShapeDtypeStruct((1, NPAD), jnp.float32),
    )

    h, hn = dense_in(x, W1, b1.reshape(1, D))
    for _ in range(4):
        accf, accd = sc_layer(hn, h, rows, cols, rowsd, consts)
        den = dsum(accd.reshape(NC * NS, NPAD)).reshape(NPAD, 1)[:N]
        h, hn = combine(accf, den)
    return dense_out(h, W2, b2.reshape(1, D))
